# baseline (speedup 1.0000x reference)
import functools

import numpy as np
import jax
import jax.numpy as jnp
from jax import lax
from jax.experimental import pallas as pl
from jax.experimental.pallas import tpu as pltpu
from jax.experimental.pallas import tpu_sc as plsc

N_LEVELS = 16
LEVEL_DIM = 2
BASE_RES = 16
HASHMAP_SIZE = 1 << 19
MASK = HASHMAP_SIZE - 1
VOXEL_SIZE = 0.01
DIM_MAX = 2.0
SIGMOID_ALPHA = 5.0 / DIM_MAX
RESOLUTION = int(DIM_MAX / VOXEL_SIZE)
PER_LEVEL_SCALE = float(np.exp2(np.log2(RESOLUTION / BASE_RES) / (N_LEVELS - 1)))
P1 = int(np.int32(np.uint32(2654435761)))
P2 = int(np.int32(np.uint32(805459861)))

LEVEL_RES = [int(np.floor(BASE_RES * PER_LEVEL_SCALE**l)) for l in range(N_LEVELS)]
LEVEL_DENSE = [(r + 1) ** 3 <= HASHMAP_SIZE for r in LEVEL_RES]

NTILES = 32
B = 128
NSLOT = N_LEVELS * 8
LANES = 16


def _sigmoid(v):
    return 1.0 / (1.0 + jnp.exp(v * (-SIGMOID_ALPHA)))


def _sdf_body(x0h, x1h, x2h, tabh, wbh, outh,
              xv, wbv, idx0v, idx1v, wv, t0v, t1v, outv, sem,
              *, pts_per_tile, nchunk):
    wid = lax.axis_index("s") * 2 + lax.axis_index("c")
    pltpu.sync_copy(wbh, wbv)
    b0v = wbv[2 * N_LEVELS, pl.ds(0, LANES)]
    base_pt = wid * pts_per_tile

    @pl.loop(0, nchunk)
    def _chunk(c):
        off = base_pt + c * B

        pltpu.sync_copy(x0h.at[pl.ds(off, B)], xv.at[0])
        pltpu.sync_copy(x1h.at[pl.ds(off, B)], xv.at[1])
        pltpu.sync_copy(x2h.at[pl.ds(off, B)], xv.at[2])

        @pl.loop(0, B // LANES)
        def _produce(i):
            o0 = i * (NSLOT * LANES)
            p0 = _sigmoid(xv[0, pl.ds(i * LANES, LANES)])
            p1 = _sigmoid(xv[1, pl.ds(i * LANES, LANES)])
            p2 = _sigmoid(xv[2, pl.ds(i * LANES, LANES)])
            for l in range(N_LEVELS):
                res = LEVEL_RES[l]
                s = res + 1
                pos0 = p0 * float(res)
                pos1 = p1 * float(res)
                pos2 = p2 * float(res)
                i0 = pos0.astype(jnp.int32)
                i1 = pos1.astype(jnp.int32)
                i2 = pos2.astype(jnp.int32)
                f0 = pos0 - i0.astype(jnp.float32)
                f1 = pos1 - i1.astype(jnp.float32)
                f2 = pos2 - i2.astype(jnp.float32)
                g0 = 1.0 - f0
                g1 = 1.0 - f1
                g2 = 1.0 - f2
                if LEVEL_DENSE[l]:
                    base = i0 + i1 * s + i2 * (s * s) + l * HASHMAP_SIZE
                    idx8 = [base + (bx + by * s + bz * (s * s))
                            for bz in (0, 1) for by in (0, 1) for bx in (0, 1)]
                else:
                    hx = [i0, i0 + 1]
                    hy0 = i1 * P1
                    hy = [hy0, hy0 + P1]
                    hz0 = i2 * P2
                    hz = [hz0, hz0 + P2]
                    idx8 = [((hx[bx] ^ hy[by] ^ hz[bz]) & MASK) + l * HASHMAP_SIZE
                            for bz in (0, 1) for by in (0, 1) for bx in (0, 1)]
                wx = (g0, f0)
                wy = (g1, f1)
                wz = (g2, f2)
                wxy = [[wx[0] * wy[0], wx[0] * wy[1]],
                       [wx[1] * wy[0], wx[1] * wy[1]]]
                corner = 0
                for bz in (0, 1):
                    for by in (0, 1):
                        for bx in (0, 1):
                            o = o0 + (l * 8 + corner) * LANES
                            e0 = idx8[corner] * 2
                            idx0v[pl.ds(o, LANES)] = e0
                            idx1v[pl.ds(o, LANES)] = e0 + 1
                            wv[pl.ds(o, LANES)] = wxy[bx][by] * wz[bz]
                            corner += 1

        d0 = pltpu.async_copy(tabh.at[idx0v], t0v, sem)
        d1 = pltpu.async_copy(tabh.at[idx1v], t1v, sem)
        d0.wait()
        d1.wait()

        @pl.loop(0, B // LANES)
        def _consume(i):
            o0 = i * (NSLOT * LANES)
            acc = jnp.zeros((LANES,), jnp.float32)
            for l in range(N_LEVELS):
                u0 = wbv[2 * l, pl.ds(0, LANES)]
                u1 = wbv[2 * l + 1, pl.ds(0, LANES)]
                for corner in range(8):
                    o = o0 + (l * 8 + corner) * LANES
                    t0 = t0v[pl.ds(o, LANES)]
                    t1 = t1v[pl.ds(o, LANES)]
                    wc = wv[pl.ds(o, LANES)]
                    acc = acc + wc * (t0 * u0 + t1 * u1)
            outv[pl.ds(i * LANES, LANES)] = acc + b0v

        pltpu.sync_copy(outv, outh.at[pl.ds(off, B)])


def kernel(x, table, W_sdf, b_sdf):
    n = x.shape[0]
    npad = -(-n // (NTILES * B)) * (NTILES * B)
    pts_per_tile = npad // NTILES
    nchunk = pts_per_tile // B

    xp = jnp.pad(x, ((0, npad - n), (0, 0)))
    x0, x1, x2 = xp[:, 0], xp[:, 1], xp[:, 2]
    tab = table.reshape(N_LEVELS * HASHMAP_SIZE * LEVEL_DIM)
    wb = jnp.repeat(
        jnp.concatenate([W_sdf[0], b_sdf[:1]])[:, None], LANES, axis=1)

    body = functools.partial(_sdf_body, pts_per_tile=pts_per_tile, nchunk=nchunk)
    out = pl.kernel(
        body,
        out_type=jax.ShapeDtypeStruct((npad,), jnp.float32),
        mesh=plsc.VectorSubcoreMesh(core_axis_name="c", subcore_axis_name="s"),
        scratch_types=[
            pltpu.VMEM((3, B), jnp.float32),
            pltpu.VMEM((2 * N_LEVELS + 1, LANES), jnp.float32),
            pltpu.VMEM((B * NSLOT,), jnp.int32),
            pltpu.VMEM((B * NSLOT,), jnp.int32),
            pltpu.VMEM((B * NSLOT,), jnp.float32),
            pltpu.VMEM((B * NSLOT,), jnp.float32),
            pltpu.VMEM((B * NSLOT,), jnp.float32),
            pltpu.VMEM((B,), jnp.float32),
            pltpu.SemaphoreType.DMA,
        ],
    )(x0, x1, x2, tab, wb)
    return out[:n].reshape(n, 1)


# speedup vs baseline: 1.0013x; 1.0013x over previous
import functools

import numpy as np
import jax
import jax.numpy as jnp
from jax import lax
from jax.experimental import pallas as pl
from jax.experimental.pallas import tpu as pltpu
from jax.experimental.pallas import tpu_sc as plsc

N_LEVELS = 16
LEVEL_DIM = 2
BASE_RES = 16
HASHMAP_SIZE = 1 << 19
MASK = HASHMAP_SIZE - 1
VOXEL_SIZE = 0.01
DIM_MAX = 2.0
SIGMOID_ALPHA = 5.0 / DIM_MAX
RESOLUTION = int(DIM_MAX / VOXEL_SIZE)
PER_LEVEL_SCALE = float(np.exp2(np.log2(RESOLUTION / BASE_RES) / (N_LEVELS - 1)))
P1 = int(np.int32(np.uint32(2654435761)))
P2 = int(np.int32(np.uint32(805459861)))

LEVEL_RES = [int(np.floor(BASE_RES * PER_LEVEL_SCALE**l)) for l in range(N_LEVELS)]
LEVEL_DENSE = [(r + 1) ** 3 <= HASHMAP_SIZE for r in LEVEL_RES]

NTILES = 32
B = 128
NSLOT = N_LEVELS * 8
LANES = 16


def _sigmoid(v):
    return 1.0 / (1.0 + jnp.exp(v * (-SIGMOID_ALPHA)))


def _sdf_body(x0h, x1h, x2h, tabh, wbh, outh,
              xv, wbv, idx0v, idx1v, wv, t0v, t1v, outv, sem,
              *, pts_per_tile, nchunk):
    wid = lax.axis_index("s") * 2 + lax.axis_index("c")
    pltpu.sync_copy(wbh, wbv)
    b0v = wbv[2 * N_LEVELS, pl.ds(0, LANES)]
    base_pt = wid * pts_per_tile

    @pl.loop(0, nchunk)
    def _chunk(c):
        off = base_pt + c * B

        pltpu.sync_copy(x0h.at[pl.ds(off, B)], xv.at[0])
        pltpu.sync_copy(x1h.at[pl.ds(off, B)], xv.at[1])
        pltpu.sync_copy(x2h.at[pl.ds(off, B)], xv.at[2])

        @pl.loop(0, B // LANES)
        def _produce(i):
            o0 = i * (NSLOT * LANES)
            p0 = _sigmoid(xv[0, pl.ds(i * LANES, LANES)])
            p1 = _sigmoid(xv[1, pl.ds(i * LANES, LANES)])
            p2 = _sigmoid(xv[2, pl.ds(i * LANES, LANES)])
            for l in range(N_LEVELS):
                res = LEVEL_RES[l]
                s = res + 1
                pos0 = p0 * float(res)
                pos1 = p1 * float(res)
                pos2 = p2 * float(res)
                i0 = pos0.astype(jnp.int32)
                i1 = pos1.astype(jnp.int32)
                i2 = pos2.astype(jnp.int32)
                f0 = pos0 - i0.astype(jnp.float32)
                f1 = pos1 - i1.astype(jnp.float32)
                f2 = pos2 - i2.astype(jnp.float32)
                g0 = 1.0 - f0
                g1 = 1.0 - f1
                g2 = 1.0 - f2
                if LEVEL_DENSE[l]:
                    base = i0 + i1 * s + i2 * (s * s) + l * HASHMAP_SIZE
                    idx8 = [base + (bx + by * s + bz * (s * s))
                            for bz in (0, 1) for by in (0, 1) for bx in (0, 1)]
                else:
                    hx = [i0, i0 + 1]
                    hy0 = i1 * P1
                    hy = [hy0, hy0 + P1]
                    hz0 = i2 * P2
                    hz = [hz0, hz0 + P2]
                    idx8 = [((hx[bx] ^ hy[by] ^ hz[bz]) & MASK) + l * HASHMAP_SIZE
                            for bz in (0, 1) for by in (0, 1) for bx in (0, 1)]
                wx = (g0, f0)
                wy = (g1, f1)
                wz = (g2, f2)
                wxy = [[wx[0] * wy[0], wx[0] * wy[1]],
                       [wx[1] * wy[0], wx[1] * wy[1]]]
                corner = 0
                for bz in (0, 1):
                    for by in (0, 1):
                        for bx in (0, 1):
                            o = o0 + (l * 8 + corner) * LANES
                            e0 = idx8[corner] * 2
                            idx0v[pl.ds(o, LANES)] = e0
                            idx1v[pl.ds(o, LANES)] = e0 + 1
                            wv[pl.ds(o, LANES)] = wxy[bx][by] * wz[bz]
                            corner += 1

        nsplit = 4
        seg = (B * NSLOT) // nsplit
        ds_ = []
        for k in range(nsplit):
            sl = pl.ds(k * seg, seg)
            ds_.append(pltpu.async_copy(tabh.at[idx0v.at[sl]], t0v.at[sl], sem))
            ds_.append(pltpu.async_copy(tabh.at[idx1v.at[sl]], t1v.at[sl], sem))
        for d in ds_:
            d.wait()

        @pl.loop(0, B // LANES)
        def _consume(i):
            o0 = i * (NSLOT * LANES)
            acc = jnp.zeros((LANES,), jnp.float32)
            for l in range(N_LEVELS):
                u0 = wbv[2 * l, pl.ds(0, LANES)]
                u1 = wbv[2 * l + 1, pl.ds(0, LANES)]
                for corner in range(8):
                    o = o0 + (l * 8 + corner) * LANES
                    t0 = t0v[pl.ds(o, LANES)]
                    t1 = t1v[pl.ds(o, LANES)]
                    wc = wv[pl.ds(o, LANES)]
                    acc = acc + wc * (t0 * u0 + t1 * u1)
            outv[pl.ds(i * LANES, LANES)] = acc + b0v

        pltpu.sync_copy(outv, outh.at[pl.ds(off, B)])


def kernel(x, table, W_sdf, b_sdf):
    n = x.shape[0]
    npad = -(-n // (NTILES * B)) * (NTILES * B)
    pts_per_tile = npad // NTILES
    nchunk = pts_per_tile // B

    xp = jnp.pad(x, ((0, npad - n), (0, 0)))
    x0, x1, x2 = xp[:, 0], xp[:, 1], xp[:, 2]
    tab = table.reshape(N_LEVELS * HASHMAP_SIZE * LEVEL_DIM)
    wb = jnp.repeat(
        jnp.concatenate([W_sdf[0], b_sdf[:1]])[:, None], LANES, axis=1)

    body = functools.partial(_sdf_body, pts_per_tile=pts_per_tile, nchunk=nchunk)
    out = pl.kernel(
        body,
        out_type=jax.ShapeDtypeStruct((npad,), jnp.float32),
        mesh=plsc.VectorSubcoreMesh(core_axis_name="c", subcore_axis_name="s"),
        scratch_types=[
            pltpu.VMEM((3, B), jnp.float32),
            pltpu.VMEM((2 * N_LEVELS + 1, LANES), jnp.float32),
            pltpu.VMEM((B * NSLOT,), jnp.int32),
            pltpu.VMEM((B * NSLOT,), jnp.int32),
            pltpu.VMEM((B * NSLOT,), jnp.float32),
            pltpu.VMEM((B * NSLOT,), jnp.float32),
            pltpu.VMEM((B * NSLOT,), jnp.float32),
            pltpu.VMEM((B,), jnp.float32),
            pltpu.SemaphoreType.DMA,
        ],
    )(x0, x1, x2, tab, wb)
    return out[:n].reshape(n, 1)


# speedup vs baseline: 5.4751x; 5.4682x over previous
import functools

import numpy as np
import jax
import jax.numpy as jnp
from jax import lax
from jax.experimental import pallas as pl
from jax.experimental.pallas import tpu as pltpu
from jax.experimental.pallas import tpu_sc as plsc

N_LEVELS = 16
LEVEL_DIM = 2
BASE_RES = 16
HASHMAP_SIZE = 1 << 19
MASK = HASHMAP_SIZE - 1
VOXEL_SIZE = 0.01
DIM_MAX = 2.0
SIGMOID_ALPHA = 5.0 / DIM_MAX
RESOLUTION = int(DIM_MAX / VOXEL_SIZE)
PER_LEVEL_SCALE = float(np.exp2(np.log2(RESOLUTION / BASE_RES) / (N_LEVELS - 1)))
P1 = int(np.int32(np.uint32(2654435761)))
P2 = int(np.int32(np.uint32(805459861)))

LEVEL_RES = [int(np.floor(BASE_RES * PER_LEVEL_SCALE**l)) for l in range(N_LEVELS)]
LEVEL_DENSE = [(r + 1) ** 3 <= HASHMAP_SIZE for r in LEVEL_RES]

NTILES = 32
B = 96
NSLOT = N_LEVELS * 8
LANES = 16


def _sigmoid(v):
    return 1.0 / (1.0 + jnp.exp(v * (-SIGMOID_ALPHA)))


def _sdf_body(x0h, x1h, x2h, t0h, t1h, wbh, outh,
              xv, wbv, idxA, idxB, wvA, wvB, t0A, t0B, t1A, t1B, outv, sem,
              *, pts_per_tile, nchunk):
    wid = lax.axis_index("s") * 2 + lax.axis_index("c")
    pltpu.sync_copy(wbh, wbv)
    b0v = wbv[2 * N_LEVELS, pl.ds(0, LANES)]
    base_pt = wid * pts_per_tile

    def produce(c, idxv, wv):
        off = base_pt + c * B
        pltpu.sync_copy(x0h.at[pl.ds(off, B)], xv.at[0])
        pltpu.sync_copy(x1h.at[pl.ds(off, B)], xv.at[1])
        pltpu.sync_copy(x2h.at[pl.ds(off, B)], xv.at[2])

        @pl.loop(0, B // LANES)
        def _produce(i):
            o0 = i * (NSLOT * LANES)
            p0 = _sigmoid(xv[0, pl.ds(i * LANES, LANES)])
            p1 = _sigmoid(xv[1, pl.ds(i * LANES, LANES)])
            p2 = _sigmoid(xv[2, pl.ds(i * LANES, LANES)])
            for l in range(N_LEVELS):
                res = LEVEL_RES[l]
                s = res + 1
                pos0 = p0 * float(res)
                pos1 = p1 * float(res)
                pos2 = p2 * float(res)
                i0 = pos0.astype(jnp.int32)
                i1 = pos1.astype(jnp.int32)
                i2 = pos2.astype(jnp.int32)
                f0 = pos0 - i0.astype(jnp.float32)
                f1 = pos1 - i1.astype(jnp.float32)
                f2 = pos2 - i2.astype(jnp.float32)
                g0 = 1.0 - f0
                g1 = 1.0 - f1
                g2 = 1.0 - f2
                if LEVEL_DENSE[l]:
                    base = i0 + i1 * s + i2 * (s * s) + l * HASHMAP_SIZE
                    idx8 = [base + (bx + by * s + bz * (s * s))
                            for bz in (0, 1) for by in (0, 1) for bx in (0, 1)]
                else:
                    hx = [i0, i0 + 1]
                    hy0 = i1 * P1
                    hy = [hy0, hy0 + P1]
                    hz0 = i2 * P2
                    hz = [hz0, hz0 + P2]
                    idx8 = [((hx[bx] ^ hy[by] ^ hz[bz]) & MASK) + l * HASHMAP_SIZE
                            for bz in (0, 1) for by in (0, 1) for bx in (0, 1)]
                wx = (g0, f0)
                wy = (g1, f1)
                wz = (g2, f2)
                wxy = [[wx[0] * wy[0], wx[0] * wy[1]],
                       [wx[1] * wy[0], wx[1] * wy[1]]]
                corner = 0
                for bz in (0, 1):
                    for by in (0, 1):
                        for bx in (0, 1):
                            o = o0 + (l * 8 + corner) * LANES
                            idxv[pl.ds(o, LANES)] = idx8[corner]
                            wv[pl.ds(o, LANES)] = wxy[bx][by] * wz[bz]
                            corner += 1

    def fire(idxv, t0v, t1v):
        d0 = pltpu.async_copy(t0h.at[idxv], t0v, sem)
        d1 = pltpu.async_copy(t1h.at[idxv], t1v, sem)
        return d0, d1

    def consume(c, wv, t0v, t1v):
        @pl.loop(0, B // LANES)
        def _consume(i):
            o0 = i * (NSLOT * LANES)
            acc = jnp.zeros((LANES,), jnp.float32)
            for l in range(N_LEVELS):
                u0 = wbv[2 * l, pl.ds(0, LANES)]
                u1 = wbv[2 * l + 1, pl.ds(0, LANES)]
                for corner in range(8):
                    o = o0 + (l * 8 + corner) * LANES
                    t0 = t0v[pl.ds(o, LANES)]
                    t1 = t1v[pl.ds(o, LANES)]
                    wc = wv[pl.ds(o, LANES)]
                    acc = acc + wc * (t0 * u0 + t1 * u1)
            outv[pl.ds(i * LANES, LANES)] = acc + b0v

        off = base_pt + c * B
        pltpu.sync_copy(outv, outh.at[pl.ds(off, B)])

    produce(0, idxA, wvA)

    @pl.loop(0, nchunk // 2)
    def _chunk(k):
        c = 2 * k
        dA = fire(idxA, t0A, t1A)
        produce(c + 1, idxB, wvB)
        for d in dA:
            d.wait()
        dB = fire(idxB, t0B, t1B)
        consume(c, wvA, t0A, t1A)
        produce(c + 2, idxA, wvA)
        for d in dB:
            d.wait()
        consume(c + 1, wvB, t0B, t1B)


def kernel(x, table, W_sdf, b_sdf):
    n = x.shape[0]
    npad = -(-n // (NTILES * B * 2)) * (NTILES * B * 2)
    pts_per_tile = npad // NTILES
    nchunk = pts_per_tile // B

    xp = jnp.pad(x, ((0, npad + B - n), (0, 0)))
    x0, x1, x2 = xp[:, 0], xp[:, 1], xp[:, 2]
    t0tab = table[..., 0].reshape(N_LEVELS * HASHMAP_SIZE)
    t1tab = table[..., 1].reshape(N_LEVELS * HASHMAP_SIZE)
    wb = jnp.repeat(
        jnp.concatenate([W_sdf[0], b_sdf[:1]])[:, None], LANES, axis=1)

    body = functools.partial(_sdf_body, pts_per_tile=pts_per_tile, nchunk=nchunk)
    out = pl.kernel(
        body,
        out_type=jax.ShapeDtypeStruct((npad,), jnp.float32),
        mesh=plsc.VectorSubcoreMesh(core_axis_name="c", subcore_axis_name="s"),
        scratch_types=[
            pltpu.VMEM((3, B), jnp.float32),
            pltpu.VMEM((2 * N_LEVELS + 1, LANES), jnp.float32),
            pltpu.VMEM((B * NSLOT,), jnp.int32),
            pltpu.VMEM((B * NSLOT,), jnp.int32),
            pltpu.VMEM((B * NSLOT,), jnp.float32),
            pltpu.VMEM((B * NSLOT,), jnp.float32),
            pltpu.VMEM((B * NSLOT,), jnp.float32),
            pltpu.VMEM((B * NSLOT,), jnp.float32),
            pltpu.VMEM((B * NSLOT,), jnp.float32),
            pltpu.VMEM((B * NSLOT,), jnp.float32),
            pltpu.VMEM((B,), jnp.float32),
            pltpu.SemaphoreType.DMA,
        ],
    )(x0, x1, x2, t0tab, t1tab, wb)
    return out[:n].reshape(n, 1)


# speedup vs baseline: 9.4206x; 1.7206x over previous
import functools

import numpy as np
import jax
import jax.numpy as jnp
from jax import lax
from jax.experimental import pallas as pl
from jax.experimental.pallas import tpu as pltpu
from jax.experimental.pallas import tpu_sc as plsc

N_LEVELS = 16
LEVEL_DIM = 2
BASE_RES = 16
HASHMAP_SIZE = 1 << 19
MASK = HASHMAP_SIZE - 1
VOXEL_SIZE = 0.01
DIM_MAX = 2.0
SIGMOID_ALPHA = 5.0 / DIM_MAX
RESOLUTION = int(DIM_MAX / VOXEL_SIZE)
PER_LEVEL_SCALE = float(np.exp2(np.log2(RESOLUTION / BASE_RES) / (N_LEVELS - 1)))
P1 = int(np.int32(np.uint32(2654435761)))
P2 = int(np.int32(np.uint32(805459861)))

LEVEL_RES = [int(np.floor(BASE_RES * PER_LEVEL_SCALE**l)) for l in range(N_LEVELS)]
LEVEL_DENSE = [(r + 1) ** 3 <= HASHMAP_SIZE for r in LEVEL_RES]

NTILES = 32
B = 96
NSLOT = N_LEVELS * 8
LANES = 16


def _sigmoid(v):
    return 1.0 / (1.0 + jnp.exp(v * (-SIGMOID_ALPHA)))


def _sdf_body(x0h, x1h, x2h, tph, wbh, outh,
              xv, wbv, idxA, idxB, wvA, wvB, tpA, tpB, outv, sem,
              *, pts_per_tile, nchunk):
    wid = lax.axis_index("s") * 2 + lax.axis_index("c")
    pltpu.sync_copy(wbh, wbv)
    b0v = wbv[2 * N_LEVELS, pl.ds(0, LANES)]
    base_pt = wid * pts_per_tile

    def produce(c, idxv, wv):
        off = base_pt + c * B
        pltpu.sync_copy(x0h.at[pl.ds(off, B)], xv.at[0])
        pltpu.sync_copy(x1h.at[pl.ds(off, B)], xv.at[1])
        pltpu.sync_copy(x2h.at[pl.ds(off, B)], xv.at[2])

        @pl.loop(0, B // LANES)
        def _produce(i):
            o0 = i * (NSLOT * LANES)
            p0 = _sigmoid(xv[0, pl.ds(i * LANES, LANES)])
            p1 = _sigmoid(xv[1, pl.ds(i * LANES, LANES)])
            p2 = _sigmoid(xv[2, pl.ds(i * LANES, LANES)])
            for l in range(N_LEVELS):
                res = LEVEL_RES[l]
                s = res + 1
                pos0 = p0 * float(res)
                pos1 = p1 * float(res)
                pos2 = p2 * float(res)
                i0 = pos0.astype(jnp.int32)
                i1 = pos1.astype(jnp.int32)
                i2 = pos2.astype(jnp.int32)
                f0 = pos0 - i0.astype(jnp.float32)
                f1 = pos1 - i1.astype(jnp.float32)
                f2 = pos2 - i2.astype(jnp.float32)
                g0 = 1.0 - f0
                g1 = 1.0 - f1
                g2 = 1.0 - f2
                if LEVEL_DENSE[l]:
                    base = i0 + i1 * s + i2 * (s * s) + l * HASHMAP_SIZE
                    idx8 = [base + (bx + by * s + bz * (s * s))
                            for bz in (0, 1) for by in (0, 1) for bx in (0, 1)]
                else:
                    hx = [i0, i0 + 1]
                    hy0 = i1 * P1
                    hy = [hy0, hy0 + P1]
                    hz0 = i2 * P2
                    hz = [hz0, hz0 + P2]
                    idx8 = [((hx[bx] ^ hy[by] ^ hz[bz]) & MASK) + l * HASHMAP_SIZE
                            for bz in (0, 1) for by in (0, 1) for bx in (0, 1)]
                wx = (g0, f0)
                wy = (g1, f1)
                wz = (g2, f2)
                wxy = [[wx[0] * wy[0], wx[0] * wy[1]],
                       [wx[1] * wy[0], wx[1] * wy[1]]]
                corner = 0
                for bz in (0, 1):
                    for by in (0, 1):
                        for bx in (0, 1):
                            o = o0 + (l * 8 + corner) * LANES
                            idxv[pl.ds(o, LANES)] = idx8[corner]
                            wv[pl.ds(o, LANES)] = wxy[bx][by] * wz[bz]
                            corner += 1

    def fire(idxv, tpv):
        return (pltpu.async_copy(tph.at[idxv], tpv, sem),)

    def consume(c, wv, tpv):
        @pl.loop(0, B // LANES)
        def _consume(i):
            o0 = i * (NSLOT * LANES)
            acc = jnp.zeros((LANES,), jnp.float32)
            for l in range(N_LEVELS):
                u0 = wbv[2 * l, pl.ds(0, LANES)]
                u1 = wbv[2 * l + 1, pl.ds(0, LANES)]
                for corner in range(8):
                    o = o0 + (l * 8 + corner) * LANES
                    tp = tpv[pl.ds(o, LANES)]
                    t0 = lax.bitcast_convert_type(tp << 16, jnp.float32)
                    t1 = lax.bitcast_convert_type(tp & (-65536), jnp.float32)
                    wc = wv[pl.ds(o, LANES)]
                    acc = acc + wc * (t0 * u0 + t1 * u1)
            outv[pl.ds(i * LANES, LANES)] = acc + b0v

        off = base_pt + c * B
        pltpu.sync_copy(outv, outh.at[pl.ds(off, B)])

    produce(0, idxA, wvA)

    @pl.loop(0, nchunk // 2)
    def _chunk(k):
        c = 2 * k
        dA = fire(idxA, tpA)
        produce(c + 1, idxB, wvB)
        for d in dA:
            d.wait()
        dB = fire(idxB, tpB)
        consume(c, wvA, tpA)
        produce(c + 2, idxA, wvA)
        for d in dB:
            d.wait()
        consume(c + 1, wvB, tpB)


def kernel(x, table, W_sdf, b_sdf):
    n = x.shape[0]
    npad = -(-n // (NTILES * B * 2)) * (NTILES * B * 2)
    pts_per_tile = npad // NTILES
    nchunk = pts_per_tile // B

    xp = jnp.pad(x, ((0, npad + B - n), (0, 0)))
    x0, x1, x2 = xp[:, 0], xp[:, 1], xp[:, 2]
    t0b = jax.lax.bitcast_convert_type(
        table[..., 0].reshape(N_LEVELS * HASHMAP_SIZE).astype(jnp.bfloat16),
        jnp.uint16).astype(jnp.uint32)
    t1b = jax.lax.bitcast_convert_type(
        table[..., 1].reshape(N_LEVELS * HASHMAP_SIZE).astype(jnp.bfloat16),
        jnp.uint16).astype(jnp.uint32)
    tpack = jax.lax.bitcast_convert_type(t0b | (t1b << 16), jnp.int32)
    wb = jnp.repeat(
        jnp.concatenate([W_sdf[0], b_sdf[:1]])[:, None], LANES, axis=1)

    body = functools.partial(_sdf_body, pts_per_tile=pts_per_tile, nchunk=nchunk)
    out = pl.kernel(
        body,
        out_type=jax.ShapeDtypeStruct((npad,), jnp.float32),
        mesh=plsc.VectorSubcoreMesh(core_axis_name="c", subcore_axis_name="s"),
        scratch_types=[
            pltpu.VMEM((3, B), jnp.float32),
            pltpu.VMEM((2 * N_LEVELS + 1, LANES), jnp.float32),
            pltpu.VMEM((B * NSLOT,), jnp.int32),
            pltpu.VMEM((B * NSLOT,), jnp.int32),
            pltpu.VMEM((B * NSLOT,), jnp.float32),
            pltpu.VMEM((B * NSLOT,), jnp.float32),
            pltpu.VMEM((B * NSLOT,), jnp.int32),
            pltpu.VMEM((B * NSLOT,), jnp.int32),
            pltpu.VMEM((B,), jnp.float32),
            pltpu.SemaphoreType.DMA,
        ],
    )(x0, x1, x2, tpack, wb)
    return out[:n].reshape(n, 1)
